# baseline (speedup 1.0000x reference)
import functools

import jax
import jax.numpy as jnp
from jax import lax
from jax.experimental import pallas as pl
from jax.experimental.pallas import tpu as pltpu
from jax.experimental.pallas import tpu_sc as plsc

N = 10000
E = 320000
D = 128
H = 64
G = 16

NC = 2
NS = 16
NW = NC * NS
EPW = E // NW
CH = 80
NCH = EPW // CH
NB = 6
PF = NB // 2
HP = H // 2
RPS = 624
TAIL = N - NS * RPS
ZR = 104
HV = H // 16


def _pack_rows(res):
    def rne(v):
        bits = lax.bitcast_convert_type(v, jnp.int32)
        return bits + 0x7FFF + lax.bitwise_and(
            lax.shift_right_logical(bits, 16), 1)

    lo16 = lax.shift_right_logical(rne(res[:, :HP]), 16)
    hi32 = lax.bitwise_and(rne(res[:, HP:]), -65536)
    return lax.bitwise_or(hi32, lo16)


def _tc_matmul1(x, W1):
    def body(x_ref, w_ref, o_ref):
        res = jnp.dot(x_ref[...], w_ref[...],
                      precision=lax.Precision.HIGHEST,
                      preferred_element_type=jnp.float32)
        o_ref[...] = _pack_rows(res)

    return pl.pallas_call(
        body,
        out_shape=jax.ShapeDtypeStruct((N, HP), jnp.int32),
    )(x, W1)


def _tc_combine_matmul(parts, b, W2):
    def body(p_ref, b_ref, w_ref, o_ref):
        h = jnp.maximum(p_ref[0] + p_ref[1] + b_ref[...], 0.0)
        res = jnp.dot(h, w_ref[...],
                      precision=lax.Precision.HIGHEST,
                      preferred_element_type=jnp.float32)
        o_ref[...] = _pack_rows(res)

    return pl.pallas_call(
        body,
        out_shape=jax.ShapeDtypeStruct((N, HP), jnp.int32),
    )(parts, b, W2)


def _tc_pool(parts, b, batch2):
    def body(p_ref, b_ref, ids_ref, o_ref):
        h = jnp.maximum(p_ref[0] + p_ref[1] + b_ref[...], 0.0)
        ids = ids_ref[...]
        gids = lax.broadcasted_iota(jnp.int32, (G, N), 0)
        onehot = (ids == gids).astype(jnp.float32)
        ssum = jnp.dot(onehot, h,
                       precision=lax.Precision.HIGHEST,
                       preferred_element_type=jnp.float32)
        cnt = jnp.sum(onehot, axis=1, keepdims=True)
        o_ref[...] = ssum / jnp.maximum(cnt, 1.0)

    return pl.pallas_call(
        body,
        out_shape=jax.ShapeDtypeStruct((G, H), jnp.float32),
    )(parts, b, batch2)


def _sc_aggregate(hpk, ei4, w2):
    mesh = plsc.VectorSubcoreMesh(core_axis_name="c", subcore_axis_name="s")

    @functools.partial(
        pl.kernel,
        out_type=jax.ShapeDtypeStruct((NC, N, H), jnp.float32),
        mesh=mesh,
        compiler_params=pltpu.CompilerParams(use_tc_tiling_on_sc=False,
                                             needs_layout_passes=False),
        scratch_types=[
            pltpu.VMEM((NCH, CH), jnp.int32),
            pltpu.VMEM((NCH, CH), jnp.int32),
            pltpu.VMEM((NCH, CH), jnp.float32),
            pltpu.VMEM((NB, CH, HP), jnp.int32),
            pltpu.VMEM((NB, CH, H), jnp.float32),
            pltpu.VMEM((ZR, H), jnp.float32),
            pltpu.VMEM_SHARED((N, H), jnp.float32),
        ] + [pltpu.SemaphoreType.DMA] * (2 * NB + 1),
    )
    def k(h_hbm, ei_hbm, w_hbm, out_hbm,
          src_v, dst_v, w_v, rows_b, rows_v, zbuf_v, acc, *sems):
        gsem = sems[:NB]
        ssem = sems[NB:2 * NB]
        sem = sems[2 * NB]
        c = lax.axis_index("c")
        s = lax.axis_index("s")
        wid = c * NS + s

        zeros16 = jnp.zeros((16,), jnp.float32)

        @pl.loop(0, ZR)
        def _(r):
            for q in range(HV):
                zbuf_v[r, pl.ds(q * 16, 16)] = zeros16

        for t in range(RPS // ZR):
            pltpu.sync_copy(zbuf_v, acc.at[pl.ds(s * RPS + t * ZR, ZR)])

        @pl.when(s == 0)
        def _():
            pltpu.sync_copy(zbuf_v.at[pl.ds(0, TAIL)],
                            acc.at[pl.ds(NS * RPS, TAIL)])

        pltpu.sync_copy(ei_hbm.at[0, wid], src_v)
        pltpu.sync_copy(ei_hbm.at[1, wid], dst_v)
        pltpu.sync_copy(w_hbm.at[wid], w_v)

        plsc.subcore_barrier()

        def gstart(j, b):
            pltpu.async_copy(h_hbm.at[src_v.at[j]], rows_b.at[b], gsem[b])

        def gwait(j, b):
            pltpu.make_async_copy(h_hbm.at[src_v.at[j]], rows_b.at[b],
                                  gsem[b]).wait()

        def sstart(j, b):
            pltpu.async_copy(rows_v.at[b], acc.at[dst_v.at[j]], ssem[b],
                             add=True)

        def swait(j, b):
            pltpu.make_async_copy(rows_v.at[b], acc.at[dst_v.at[j]],
                                  ssem[b]).wait()

        mask_hi = jnp.full((16,), -65536, jnp.int32)
        shift16 = jnp.full((16,), 16, jnp.int32)

        def scale(j, b):
            for g in range(CH // 16):
                w16 = w_v[j, pl.ds(g * 16, 16)]
                for l in range(16):
                    wl = w16.at[jnp.full((16,), l, jnp.int32)].get(
                        mode="promise_in_bounds")
                    r = g * 16 + l
                    for q in range(HP // 16):
                        v = rows_b[b, r, pl.ds(q * 16, 16)]
                        even = plsc.bitcast(
                            lax.shift_left(v, shift16), jnp.float32)
                        odd = plsc.bitcast(
                            lax.bitwise_and(v, mask_hi), jnp.float32)
                        rows_v[b, r, pl.ds(q * 32, 16)] = even * wl
                        rows_v[b, r, pl.ds(q * 32 + 16, 16)] = odd * wl

        def process(j, b):
            br = (b + PF) % NB

            @pl.when(j >= PF)
            def _():
                swait(j - PF, br)

            @pl.when(j + PF <= NCH - 1)
            def _():
                gstart(j + PF, br)

            gwait(j, b)
            scale(j, b)
            sstart(j, b)

        for b in range(PF):
            gstart(b, b)

        @pl.loop(0, NCH // NB)
        def _(i):
            for b in range(NB):
                process(i * NB + b, b)

        for j in range(NB * (NCH // NB), NCH):
            process(j, j % NB)
        for j in range(NCH - PF, NCH):
            swait(j, j % NB)

        plsc.subcore_barrier()

        pltpu.sync_copy(acc.at[pl.ds(s * RPS, RPS)],
                        out_hbm.at[c, pl.ds(s * RPS, RPS)])

        @pl.when(s == 0)
        def _():
            pltpu.sync_copy(acc.at[pl.ds(NS * RPS, TAIL)],
                            out_hbm.at[c, pl.ds(NS * RPS, TAIL)])

    return k(hpk, ei4, w2)


_PERM = (list(range(0, 16)) + list(range(32, 48))
         + list(range(16, 32)) + list(range(48, 64)))
_INV = [0] * H
for _k, _p in enumerate(_PERM):
    _INV[_p] = _k


def kernel(x, edge_index, edge_weight, batch, W1, b1, W2, b2):
    ei4 = edge_index.reshape(2, NW, NCH, CH)
    w2d = edge_weight.reshape(NW, NCH, CH)
    batch2 = batch.reshape(1, N)
    perm = jnp.array(_PERM, jnp.int32)

    h1p = _tc_matmul1(x, W1)
    p1 = _sc_aggregate(h1p, ei4, w2d)
    h2p = _tc_combine_matmul(p1, b1[perm].reshape(1, H), W2[perm, :])
    p2 = _sc_aggregate(h2p, ei4, w2d)
    out = _tc_pool(p2, b2[perm].reshape(1, H), batch2)
    return out[:, jnp.array(_INV, jnp.int32)]


# speedup vs baseline: 1.2688x; 1.2688x over previous
import functools

import jax
import jax.numpy as jnp
from jax import lax
from jax.experimental import pallas as pl
from jax.experimental.pallas import tpu as pltpu
from jax.experimental.pallas import tpu_sc as plsc

N = 10000
E = 320000
D = 128
H = 64
G = 16

NC = 2
NS = 16
NW = NC * NS
EPW = E // NW
CH = 80
NCH = EPW // CH
NB = 4
HP = H // 2
RPS = 624
TAIL = N - NS * RPS
ZR = 208
HV = H // 16


def _pack_rows(res):
    def rne(v):
        bits = lax.bitcast_convert_type(v, jnp.int32)
        return bits + 0x7FFF + lax.bitwise_and(
            lax.shift_right_logical(bits, 16), 1)

    lo16 = lax.shift_right_logical(rne(res[:, :HP]), 16)
    hi32 = lax.bitwise_and(rne(res[:, HP:]), -65536)
    return lax.bitwise_or(hi32, lo16)


def _tc_matmul1(x, W1):
    def body(x_ref, w_ref, o_ref):
        res = jnp.dot(x_ref[...], w_ref[...],
                      precision=lax.Precision.HIGHEST,
                      preferred_element_type=jnp.float32)
        o_ref[...] = _pack_rows(res)

    return pl.pallas_call(
        body,
        out_shape=jax.ShapeDtypeStruct((N, HP), jnp.int32),
    )(x, W1)


def _tc_combine_matmul(parts, b, W2):
    def body(p_ref, b_ref, w_ref, o_ref):
        h = jnp.maximum(p_ref[0] + p_ref[1] + b_ref[...], 0.0)
        res = jnp.dot(h, w_ref[...],
                      precision=lax.Precision.HIGHEST,
                      preferred_element_type=jnp.float32)
        o_ref[...] = _pack_rows(res)

    return pl.pallas_call(
        body,
        out_shape=jax.ShapeDtypeStruct((N, HP), jnp.int32),
    )(parts, b, W2)


def _tc_pool(parts, b, batch2):
    def body(p_ref, b_ref, ids_ref, o_ref):
        h = jnp.maximum(p_ref[0] + p_ref[1] + b_ref[...], 0.0)
        ids = ids_ref[...]
        gids = lax.broadcasted_iota(jnp.int32, (G, N), 0)
        onehot = (ids == gids).astype(jnp.float32)
        ssum = jnp.dot(onehot, h,
                       precision=lax.Precision.HIGHEST,
                       preferred_element_type=jnp.float32)
        cnt = jnp.sum(onehot, axis=1, keepdims=True)
        o_ref[...] = ssum / jnp.maximum(cnt, 1.0)

    return pl.pallas_call(
        body,
        out_shape=jax.ShapeDtypeStruct((G, H), jnp.float32),
    )(parts, b, batch2)


def _sc_aggregate(hpk, ei4, w2):
    mesh = plsc.VectorSubcoreMesh(core_axis_name="c", subcore_axis_name="s")

    @functools.partial(
        pl.kernel,
        out_type=jax.ShapeDtypeStruct((NC, N, H), jnp.float32),
        mesh=mesh,
        compiler_params=pltpu.CompilerParams(use_tc_tiling_on_sc=False,
                                             needs_layout_passes=False),
        scratch_types=[
            pltpu.VMEM((NCH, CH), jnp.int32),
            pltpu.VMEM((NCH, CH), jnp.int32),
            pltpu.VMEM((NCH, CH), jnp.float32),
            pltpu.VMEM((NB, CH, HP), jnp.int32),
            pltpu.VMEM((NB, CH, H), jnp.float32),
            pltpu.VMEM((ZR, H), jnp.float32),
            pltpu.VMEM_SHARED((N, H), jnp.float32),
        ] + [pltpu.SemaphoreType.DMA] * (2 * NB + 1),
    )
    def k(h_hbm, ei_hbm, w_hbm, out_hbm,
          src_v, dst_v, w_v, rows_b, rows_v, zbuf_v, acc, *sems):
        gsem = sems[:NB]
        ssem = sems[NB:2 * NB]
        sem = sems[2 * NB]
        c = lax.axis_index("c")
        s = lax.axis_index("s")
        wid = c * NS + s

        zeros16 = jnp.zeros((16,), jnp.float32)

        @pl.loop(0, ZR)
        def _(r):
            for q in range(HV):
                zbuf_v[r, pl.ds(q * 16, 16)] = zeros16

        for t in range(RPS // ZR):
            pltpu.sync_copy(zbuf_v, acc.at[pl.ds(s * RPS + t * ZR, ZR)])

        @pl.when(s == 0)
        def _():
            pltpu.sync_copy(zbuf_v.at[pl.ds(0, TAIL)],
                            acc.at[pl.ds(NS * RPS, TAIL)])

        pltpu.sync_copy(ei_hbm.at[0, wid], src_v)
        pltpu.sync_copy(ei_hbm.at[1, wid], dst_v)
        pltpu.sync_copy(w_hbm.at[wid], w_v)

        plsc.subcore_barrier()

        def gstart(j, b):
            pltpu.async_copy(h_hbm.at[src_v.at[j]], rows_b.at[b], gsem[b])

        def gwait(j, b):
            pltpu.make_async_copy(h_hbm.at[src_v.at[j]], rows_b.at[b],
                                  gsem[b]).wait()

        def sstart(j, b):
            pltpu.async_copy(rows_v.at[b], acc.at[dst_v.at[j]], ssem[b],
                             add=True)

        def swait(j, b):
            pltpu.make_async_copy(rows_v.at[b], acc.at[dst_v.at[j]],
                                  ssem[b]).wait()

        mask_hi = jnp.full((16,), -65536, jnp.int32)
        shift16 = jnp.full((16,), 16, jnp.int32)

        def scale(j, b):
            for g in range(CH // 16):
                w16 = w_v[j, pl.ds(g * 16, 16)]
                for l in range(16):
                    wl = w16.at[jnp.full((16,), l, jnp.int32)].get(
                        mode="promise_in_bounds")
                    r = g * 16 + l
                    for q in range(HP // 16):
                        v = rows_b[b, r, pl.ds(q * 16, 16)]
                        even = plsc.bitcast(
                            lax.shift_left(v, shift16), jnp.float32)
                        odd = plsc.bitcast(
                            lax.bitwise_and(v, mask_hi), jnp.float32)
                        rows_v[b, r, pl.ds(q * 32, 16)] = even * wl
                        rows_v[b, r, pl.ds(q * 32 + 16, 16)] = odd * wl

        def process(j, b):
            br = (b + 2) % NB

            @pl.when(j >= 2)
            def _():
                swait(j - 2, br)

            @pl.when(j + 2 <= NCH - 1)
            def _():
                gstart(j + 2, br)

            gwait(j, b)
            scale(j, b)
            sstart(j, b)

        gstart(0, 0)
        gstart(1, 1)

        @pl.loop(0, (NCH - 1) // NB)
        def _(i):
            for b in range(NB):
                process(i * NB + b, b)

        process(NCH - 1, (NCH - 1) % NB)
        swait(NCH - 2, (NCH - 2) % NB)
        swait(NCH - 1, (NCH - 1) % NB)

        plsc.subcore_barrier()

        pltpu.sync_copy(acc.at[pl.ds(s * RPS, RPS)],
                        out_hbm.at[c, pl.ds(s * RPS, RPS)])

        @pl.when(s == 0)
        def _():
            pltpu.sync_copy(acc.at[pl.ds(NS * RPS, TAIL)],
                            out_hbm.at[c, pl.ds(NS * RPS, TAIL)])

    return k(hpk, ei4, w2)


_PERM = (list(range(0, 16)) + list(range(32, 48))
         + list(range(16, 32)) + list(range(48, 64)))
_INV = [0] * H
for _k, _p in enumerate(_PERM):
    _INV[_p] = _k


def kernel(x, edge_index, edge_weight, batch, W1, b1, W2, b2):
    ei4 = edge_index.reshape(2, NW, NCH, CH)
    w2d = edge_weight.reshape(NW, NCH, CH)
    batch2 = batch.reshape(1, N)
    perm = jnp.array(_PERM, jnp.int32)

    h1p = _tc_matmul1(x, W1)
    p1 = _sc_aggregate(h1p, ei4, w2d)
    h2p = _tc_combine_matmul(p1, b1[perm].reshape(1, H), W2[perm, :])
    p2 = _sc_aggregate(h2p, ei4, w2d)
    out = _tc_pool(p2, b2[perm].reshape(1, H), batch2)
    return out[:, jnp.array(_INV, jnp.int32)]
